# baseline (speedup 1.0000x reference)
import functools

import jax
import jax.numpy as jnp
import numpy as np
from jax import lax
from jax.experimental import pallas as pl
from jax.experimental.pallas import tpu as pltpu
from jax.experimental.pallas import tpu_sc as plsc

D = 64
L_SEQ = 200
NC = 2
NS = 16
NW = NC * NS
LANES = 16


def _sinusoidal_pe_np(length, d_model):
    pos = np.arange(length, dtype=np.float32)[:, None]
    div = np.exp(
        np.arange(0, d_model, 2, dtype=np.float32) * (-np.log(10000.0) / d_model)
    )
    pe = np.zeros((length, d_model), dtype=np.float32)
    pe[:, 0::2] = np.sin(pos * div)
    pe[:, 1::2] = np.cos(pos * div)
    return pe


@functools.partial(jax.jit, static_argnames=("n_rows",))
def _embed(idx, table, pe, n_rows):
    rows_per_w = n_rows // NW
    seqs_per_w = rows_per_w // L_SEQ
    mesh = plsc.VectorSubcoreMesh(core_axis_name="c", subcore_axis_name="s")

    @functools.partial(
        pl.kernel,
        out_type=jax.ShapeDtypeStruct((n_rows, D), jnp.float32),
        mesh=mesh,
        scratch_types=[
            pltpu.VMEM((rows_per_w,), jnp.int32),
            pltpu.VMEM((L_SEQ, D), jnp.float32),
            pltpu.VMEM((2, L_SEQ, 2 * D), jnp.float32),
            pltpu.VMEM((L_SEQ, D), jnp.float32),
            pltpu.SemaphoreType.DMA,
            pltpu.SemaphoreType.DMA,
        ],
        compiler_params=pltpu.CompilerParams(
            use_tc_tiling_on_sc=True, needs_layout_passes=False
        ),
    )
    def k(table_hbm, idx_hbm, pe_hbm, out_hbm,
          idx_v, pe_v, rows_v, out_v, sem0, sem1):
        wid = lax.axis_index("s") * NC + lax.axis_index("c")
        base = wid * rows_per_w
        pltpu.sync_copy(idx_hbm.at[pl.ds(base, rows_per_w)], idx_v)
        pltpu.sync_copy(pe_hbm, pe_v)
        sems = (sem0, sem1)

        def start(s, b):
            pltpu.async_copy(
                table_hbm.at[idx_v.at[pl.ds(s * L_SEQ, L_SEQ)]],
                rows_v.at[b],
                sems[b],
            )

        def wait(s, b):
            pltpu.make_async_copy(
                table_hbm.at[idx_v.at[pl.ds(s * L_SEQ, L_SEQ)]],
                rows_v.at[b],
                sems[b],
            ).wait()

        def process(s, b):
            wait(s, b)
            rbuf = rows_v.at[b]

            def row(i, _):
                for d in range(D // LANES):
                    sl = pl.ds(d * LANES, LANES)
                    out_v[i, sl] = rbuf[i, sl] + pe_v[i, sl]
                return 0

            lax.fori_loop(0, L_SEQ, row, 0, unroll=4)
            pltpu.sync_copy(out_v, out_hbm.at[pl.ds(base + s * L_SEQ, L_SEQ)])

        start(0, 0)
        start(1, 1)

        def body(g, _):
            s = 2 * g
            process(s, 0)
            start(s + 2, 0)
            process(s + 1, 1)
            start(s + 3, 1)
            return 0

        lax.fori_loop(0, seqs_per_w // 2 - 1, body, 0)
        process(seqs_per_w - 2, 0)
        process(seqs_per_w - 1, 1)

    return k(table, idx, pe)


def kernel(sequence, token_table):
    B, L = sequence.shape
    V, d = token_table.shape
    flat = sequence.reshape(-1).astype(jnp.int32)
    padded = jnp.pad(token_table.T, ((0, 128 - d), (0, 0))).T
    pe = jnp.asarray(_sinusoidal_pe_np(L, d))
    out = _embed(flat, padded, pe, B * L)
    return out.reshape(B, L, d)


# speedup vs baseline: 1.0637x; 1.0637x over previous
import functools

import jax
import jax.numpy as jnp
import numpy as np
from jax import lax
from jax.experimental import pallas as pl
from jax.experimental.pallas import tpu as pltpu
from jax.experimental.pallas import tpu_sc as plsc

D = 64
L_SEQ = 200
NC = 2
NS = 16
NW = NC * NS
LANES = 16


def _sinusoidal_pe_np(length, d_model):
    pos = np.arange(length, dtype=np.float32)[:, None]
    div = np.exp(
        np.arange(0, d_model, 2, dtype=np.float32) * (-np.log(10000.0) / d_model)
    )
    pe = np.zeros((length, d_model), dtype=np.float32)
    pe[:, 0::2] = np.sin(pos * div)
    pe[:, 1::2] = np.cos(pos * div)
    return pe


@functools.partial(jax.jit, static_argnames=("n_rows",))
def _embed(idx, table, pe, n_rows):
    rows_per_w = n_rows // NW
    seqs_per_w = rows_per_w // L_SEQ
    mesh = plsc.VectorSubcoreMesh(core_axis_name="c", subcore_axis_name="s")

    @functools.partial(
        pl.kernel,
        out_type=jax.ShapeDtypeStruct((n_rows, D), jnp.float32),
        mesh=mesh,
        scratch_types=[
            pltpu.VMEM((rows_per_w,), jnp.int32),
            pltpu.VMEM((L_SEQ, D), jnp.float32),
            pltpu.VMEM((2, L_SEQ, 2 * D), jnp.float32),
            pltpu.VMEM((L_SEQ, D), jnp.float32),
            pltpu.SemaphoreType.DMA,
            pltpu.SemaphoreType.DMA,
        ],
        compiler_params=pltpu.CompilerParams(use_tc_tiling_on_sc=True),
    )
    def k(table_hbm, idx_hbm, pe_hbm, out_hbm,
          idx_v, pe_v, rows_v, out_v, sem0, sem1):
        wid = lax.axis_index("s") * NC + lax.axis_index("c")
        base = wid * rows_per_w
        pltpu.sync_copy(idx_hbm.at[pl.ds(base, rows_per_w)], idx_v)
        pltpu.sync_copy(pe_hbm, pe_v)
        sems = (sem0, sem1)

        def start(s, b):
            pltpu.async_copy(
                table_hbm.at[idx_v.at[pl.ds(s * L_SEQ, L_SEQ)]],
                rows_v.at[b],
                sems[b],
            )

        def wait(s, b):
            pltpu.make_async_copy(
                table_hbm.at[idx_v.at[pl.ds(s * L_SEQ, L_SEQ)]],
                rows_v.at[b],
                sems[b],
            ).wait()

        def process(s, b):
            wait(s, b)
            rbuf = rows_v.at[b]

            def row(i, _):
                for d in range(D // LANES):
                    sl = pl.ds(d * LANES, LANES)
                    out_v[i, sl] = rbuf[i, sl] + pe_v[i, sl]
                return 0

            lax.fori_loop(0, L_SEQ, row, 0)
            pltpu.sync_copy(out_v, out_hbm.at[pl.ds(base + s * L_SEQ, L_SEQ)])

        start(0, 0)
        start(1, 1)

        def body(g, _):
            s = 2 * g
            process(s, 0)
            start(s + 2, 0)
            process(s + 1, 1)
            start(s + 3, 1)
            return 0

        lax.fori_loop(0, seqs_per_w // 2 - 1, body, 0)
        process(seqs_per_w - 2, 0)
        process(seqs_per_w - 1, 1)

    return k(table, idx, pe)


def kernel(sequence, token_table):
    B, L = sequence.shape
    V, d = token_table.shape
    flat = sequence.reshape(-1).astype(jnp.int32)
    padded = jnp.pad(token_table.T, ((0, 128 - d), (0, 0))).T
    pe = jnp.asarray(_sinusoidal_pe_np(L, d))
    out = _embed(flat, padded, pe, B * L)
    return out.reshape(B, L, d)


# speedup vs baseline: 1.0646x; 1.0008x over previous
import functools

import jax
import jax.numpy as jnp
import numpy as np
from jax import lax
from jax.experimental import pallas as pl
from jax.experimental.pallas import tpu as pltpu
from jax.experimental.pallas import tpu_sc as plsc

D = 64
L_SEQ = 200
NC = 2
NS = 16
NW = NC * NS
LANES = 16


def _sinusoidal_pe_np(length, d_model):
    pos = np.arange(length, dtype=np.float32)[:, None]
    div = np.exp(
        np.arange(0, d_model, 2, dtype=np.float32) * (-np.log(10000.0) / d_model)
    )
    pe = np.zeros((length, d_model), dtype=np.float32)
    pe[:, 0::2] = np.sin(pos * div)
    pe[:, 1::2] = np.cos(pos * div)
    return pe


@functools.partial(jax.jit, static_argnames=("n_rows",))
def _embed(idx, table, pe, n_rows):
    rows_per_w = n_rows // NW
    seqs_per_w = rows_per_w // L_SEQ
    mesh = plsc.VectorSubcoreMesh(core_axis_name="c", subcore_axis_name="s")

    @functools.partial(
        pl.kernel,
        out_type=jax.ShapeDtypeStruct((n_rows, D), jnp.float32),
        mesh=mesh,
        scratch_types=[
            pltpu.VMEM((rows_per_w,), jnp.int32),
            pltpu.VMEM((L_SEQ, D), jnp.float32),
            pltpu.VMEM((2, L_SEQ, 2 * D), jnp.float32),
            pltpu.VMEM((L_SEQ, D), jnp.float32),
            pltpu.SemaphoreType.DMA,
            pltpu.SemaphoreType.DMA,
            pltpu.SemaphoreType.DMA,
        ],
        compiler_params=pltpu.CompilerParams(use_tc_tiling_on_sc=True),
    )
    def k(table_hbm, idx_hbm, pe_hbm, out_hbm,
          idx_v, pe_v, rows_v, out_v, sem0, sem1, sem_st):
        wid = lax.axis_index("s") * NC + lax.axis_index("c")
        base = wid * rows_per_w
        pltpu.sync_copy(idx_hbm.at[pl.ds(base, rows_per_w)], idx_v)
        pltpu.sync_copy(pe_hbm, pe_v)
        sems = (sem0, sem1)

        def start(s, b):
            pltpu.async_copy(
                table_hbm.at[idx_v.at[pl.ds(s * L_SEQ, L_SEQ)]],
                rows_v.at[b],
                sems[b],
            )

        def wait(s, b):
            pltpu.make_async_copy(
                table_hbm.at[idx_v.at[pl.ds(s * L_SEQ, L_SEQ)]],
                rows_v.at[b],
                sems[b],
            ).wait()

        def store_wait(s):
            pltpu.make_async_copy(
                out_v, out_hbm.at[pl.ds(base + s * L_SEQ, L_SEQ)], sem_st
            ).wait()

        def process(s, b):
            wait(s, b)

            @pl.when(s >= 1)
            def _():
                store_wait(s - 1)

            rbuf = rows_v.at[b]

            def row(i, _):
                for d in range(D // LANES):
                    sl = pl.ds(d * LANES, LANES)
                    out_v[i, sl] = rbuf[i, sl] + pe_v[i, sl]
                return 0

            lax.fori_loop(0, L_SEQ, row, 0)
            pltpu.async_copy(
                out_v, out_hbm.at[pl.ds(base + s * L_SEQ, L_SEQ)], sem_st)

        start(0, 0)
        start(1, 1)

        def body(g, _):
            s = 2 * g
            process(s, 0)
            start(s + 2, 0)
            process(s + 1, 1)
            start(s + 3, 1)
            return 0

        lax.fori_loop(0, seqs_per_w // 2 - 1, body, 0)
        process(seqs_per_w - 2, 0)
        process(seqs_per_w - 1, 1)
        store_wait(seqs_per_w - 1)

    return k(table, idx, pe)


def kernel(sequence, token_table):
    B, L = sequence.shape
    V, d = token_table.shape
    flat = sequence.reshape(-1).astype(jnp.int32)
    padded = jnp.pad(token_table.T, ((0, 128 - d), (0, 0))).T
    pe = jnp.asarray(_sinusoidal_pe_np(L, d))
    out = _embed(flat, padded, pe, B * L)
    return out.reshape(B, L, d)
